# baseline (speedup 1.0000x reference)
import functools

import jax
import jax.numpy as jnp
from jax import lax
from jax.experimental import pallas as pl
from jax.experimental.pallas import tpu as pltpu
from jax.experimental.pallas import tpu_sc as plsc

N = 10000
E = 320000
NFEAT = 128
NHID = 64
NCLASS = 16

NC = 2
NS = 16
L = 16
NW = NC * NS
EPW = E // NW
RPT = 624
TAIL_START = NS * RPT
TAIL = N - TAIL_START


def _spmm_sc(feat: int, sb: int, w_dma: int):
  W = w_dma
  chunk = sb * W
  nch = E // chunk
  assert nch * chunk == E and W % 8 == 0 and W <= 128 and chunk % L == 0
  mesh = plsc.VectorSubcoreMesh(core_axis_name="c", subcore_axis_name="s")
  NB = 3

  @functools.partial(
      pl.kernel,
      out_type=pltpu.HBM((NC, N, feat), jnp.float32),
      mesh=mesh,
      compiler_params=pltpu.CompilerParams(use_tc_tiling_on_sc=False),
      scratch_types=[
          pltpu.VMEM((NB, chunk), jnp.int32),
          pltpu.VMEM((NB, sb, W), jnp.int32),
          pltpu.VMEM((NB, chunk), jnp.float32),
          pltpu.VMEM((NB, chunk, feat), jnp.float32),
          pltpu.VMEM_SHARED((N, feat), jnp.float32),
          [pltpu.SemaphoreType.DMA] * NB,
          [pltpu.SemaphoreType.DMA] * NB,
      ],
  )
  def k(ei_hbm, w_hbm, h_hbm, out_hbm, src_v, dst_v, w_v, rows_v,
        acc, gsem, ssem):
    c = lax.axis_index("c")
    s = lax.axis_index("s")
    wid = s * NC + c

    q0 = wid * nch // NW
    q1 = (wid + 1) * nch // NW

    def fetch(q, b):
      base = q * chunk
      pltpu.sync_copy(ei_hbm.at[0, pl.ds(base, chunk)], src_v.at[b])
      for j in range(sb):
        pltpu.sync_copy(ei_hbm.at[1, pl.ds(base + j * W, W)], dst_v.at[b, j])
      pltpu.sync_copy(w_hbm.at[pl.ds(base, chunk)], w_v.at[b])
      for j in range(sb):
        pltpu.async_copy(h_hbm.at[src_v.at[b, pl.ds(j * W, W)]],
                         rows_v.at[b, pl.ds(j * W, W)], gsem[b])

    def wait_gather(b):
      for j in range(sb):
        pltpu.make_async_copy(h_hbm.at[src_v.at[b, pl.ds(j * W, W)]],
                              rows_v.at[b, pl.ds(j * W, W)], gsem[b]).wait()

    def wait_scatter(b):
      for j in range(sb):
        pltpu.make_async_copy(rows_v.at[b, pl.ds(j * W, W)],
                              acc.at[dst_v.at[b, j]], ssem[b]).wait()

    fetch(q0, 0)

    zero = jnp.zeros((L,), jnp.float32)
    zrows = min(chunk, RPT)

    def zbody(i, _):
      for j in range(feat // L):
        rows_v[NB - 1, i, pl.ds(j * L, L)] = zero
      return 0

    lax.fori_loop(0, zrows, zbody, 0)
    done = 0
    while done < RPT:
      step = min(zrows, RPT - done)
      pltpu.sync_copy(rows_v.at[NB - 1, pl.ds(0, step)],
                      acc.at[pl.ds(s * RPT + done, step)])
      done += step

    @pl.when(s == NS - 1)
    def _zero_tail():
      pltpu.sync_copy(rows_v.at[NB - 1, pl.ds(0, TAIL)],
                      acc.at[pl.ds(TAIL_START, TAIL)])

    plsc.subcore_barrier()

    def process(q, b):
      wait_gather(b)
      nb = (b + 1) % NB

      @pl.when(q - 2 >= q0)
      def _drain_prev():
        wait_scatter(nb)

      @pl.when(q + 1 < q1)
      def _prefetch():
        fetch(q + 1, nb)

      def gbody(g):
        w16 = w_v[b, pl.ds(g * L, L)]
        rowbase = g * L
        for e in range(L):
          wb = w16[jnp.full((L,), e, jnp.int32)]
          for f in range(feat // L):
            sl = pl.ds(f * L, L)
            rows_v[b, rowbase + e, sl] = rows_v[b, rowbase + e, sl] * wb

      plsc.parallel_loop(0, chunk // L, 1, unroll=4)(gbody)
      for j in range(sb):
        pltpu.async_copy(rows_v.at[b, pl.ds(j * W, W)],
                         acc.at[dst_v.at[b, j]], ssem[b], add=True)

    @pl.loop(q0, q1, step=NB)
    def _chunk_trip(i):
      for bb in range(NB):
        @pl.when(i + bb < q1)
        def _one():
          process(i + bb, bb)

    last = (q1 - 1 - q0) % NB
    for bb in range(NB):
      @pl.when(jnp.logical_or(last == bb, (last + NB - 1) % NB == bb))
      def _drain_tail():
        wait_scatter(bb)

    plsc.subcore_barrier()
    pltpu.sync_copy(acc.at[pl.ds(s * RPT, RPT)],
                    out_hbm.at[c, pl.ds(s * RPT, RPT)])

    @pl.when(s == NS - 1)
    def _write_tail():
      pltpu.sync_copy(acc.at[pl.ds(TAIL_START, TAIL)],
                      out_hbm.at[c, pl.ds(TAIL_START, TAIL)])

  return k


_spmm_hid = _spmm_sc(NHID, 5, 80)
_spmm_out = _spmm_sc(NCLASS, 10, 128)


def _mm1_body(x_ref, w_ref, o_ref):
  o_ref[...] = jnp.dot(x_ref[...], w_ref[...],
                       preferred_element_type=jnp.float32)


def _mm1(x, W1):
  return pl.pallas_call(
      _mm1_body,
      out_shape=jax.ShapeDtypeStruct((N, NHID), jnp.float32),
  )(x, W1)


def _mid_body(p_ref, b1_ref, w2_ref, o_ref):
  h = p_ref[0] + p_ref[1] + b1_ref[...]
  h = jnp.maximum(h, 0.0)
  o_ref[...] = jnp.dot(h, w2_ref[...], preferred_element_type=jnp.float32)


def _mid(parts, b1, W2):
  return pl.pallas_call(
      _mid_body,
      out_shape=jax.ShapeDtypeStruct((N, NCLASS), jnp.float32),
  )(parts, b1, W2)


def _fin_body(q_ref, b2_ref, o_ref):
  o_ref[...] = q_ref[0] + q_ref[1] + b2_ref[...]


def _fin(parts, b2):
  return pl.pallas_call(
      _fin_body,
      out_shape=jax.ShapeDtypeStruct((N, NCLASS), jnp.float32),
  )(parts, b2)


def kernel(x, edge_index, edge_weight, W1, b1, W2, b2):
  ei = edge_index.astype(jnp.int32)
  w = edge_weight.astype(jnp.float32)
  h = _mm1(x, W1)
  parts = _spmm_hid(ei, w, h)
  h2 = _mid(parts, b1.reshape(1, NHID), W2)
  parts2 = _spmm_out(ei, w, h2)
  return _fin(parts2, b2.reshape(1, NCLASS))


# speedup vs baseline: 1.3052x; 1.3052x over previous
import functools

import jax
import jax.numpy as jnp
from jax import lax
from jax.experimental import pallas as pl
from jax.experimental.pallas import tpu as pltpu
from jax.experimental.pallas import tpu_sc as plsc

N = 10000
E = 320000
NFEAT = 128
NHID = 64
NCLASS = 16

NC = 2
NS = 16
L = 16
NW = NC * NS
EPW = E // NW
RPT = 624
TAIL_START = NS * RPT
TAIL = N - TAIL_START


def _spmm_sc(feat: int, sb: int, w_dma: int):
  W = w_dma
  chunk = sb * W
  nch = E // chunk
  assert nch * chunk == E and W % 8 == 0 and W <= 128 and chunk % L == 0
  mesh = plsc.VectorSubcoreMesh(core_axis_name="c", subcore_axis_name="s")
  NB = 3

  @functools.partial(
      pl.kernel,
      out_type=pltpu.HBM((NC, N, feat), jnp.float32),
      mesh=mesh,
      compiler_params=pltpu.CompilerParams(use_tc_tiling_on_sc=False),
      scratch_types=[
          pltpu.VMEM((NB, 3, sb, W), jnp.int32),
          pltpu.VMEM((NB, chunk, feat), jnp.float32),
          pltpu.VMEM_SHARED((N, feat), jnp.float32),
          [pltpu.SemaphoreType.DMA] * NB,
          [pltpu.SemaphoreType.DMA] * NB,
      ],
  )
  def k(epk_hbm, h_hbm, out_hbm, ebuf, rows_v, acc, gsem, ssem):
    c = lax.axis_index("c")
    s = lax.axis_index("s")
    wid = s * NC + c

    q0 = wid * nch // NW
    q1 = (wid + 1) * nch // NW

    def fetch(q, b):
      pltpu.sync_copy(epk_hbm.at[:, pl.ds(q * sb, sb)], ebuf.at[b])
      for j in range(sb):
        pltpu.async_copy(h_hbm.at[ebuf.at[b, 0, j]],
                         rows_v.at[b, pl.ds(j * W, W)], gsem[b])

    def wait_gather(b):
      for j in range(sb):
        pltpu.make_async_copy(h_hbm.at[ebuf.at[b, 0, j]],
                              rows_v.at[b, pl.ds(j * W, W)], gsem[b]).wait()

    def wait_scatter(b):
      for j in range(sb):
        pltpu.make_async_copy(rows_v.at[b, pl.ds(j * W, W)],
                              acc.at[ebuf.at[b, 1, j]], ssem[b]).wait()

    fetch(q0, 0)

    zero = jnp.zeros((L,), jnp.float32)
    zrows = min(chunk, RPT)

    def zbody(i, _):
      for j in range(feat // L):
        rows_v[NB - 1, i, pl.ds(j * L, L)] = zero
      return 0

    lax.fori_loop(0, zrows, zbody, 0)
    done = 0
    while done < RPT:
      step = min(zrows, RPT - done)
      pltpu.sync_copy(rows_v.at[NB - 1, pl.ds(0, step)],
                      acc.at[pl.ds(s * RPT + done, step)])
      done += step

    @pl.when(s == NS - 1)
    def _zero_tail():
      pltpu.sync_copy(rows_v.at[NB - 1, pl.ds(0, TAIL)],
                      acc.at[pl.ds(TAIL_START, TAIL)])

    plsc.subcore_barrier()

    def process(q, b):
      wait_gather(b)
      nb = (b + 1) % NB

      @pl.when(q - 2 >= q0)
      def _drain_prev():
        wait_scatter(nb)

      @pl.when(q + 1 < q1)
      def _prefetch():
        fetch(q + 1, nb)

      def gbody(g):
        j = g // (W // L)
        w16 = lax.bitcast_convert_type(
            ebuf[b, 2, j, pl.ds((g % (W // L)) * L, L)], jnp.float32)
        rowbase = g * L
        for e in range(L):
          wb = w16[jnp.full((L,), e, jnp.int32)]
          for f in range(feat // L):
            sl = pl.ds(f * L, L)
            rows_v[b, rowbase + e, sl] = rows_v[b, rowbase + e, sl] * wb

      plsc.parallel_loop(0, chunk // L, 1, unroll=4)(gbody)
      for j in range(sb):
        pltpu.async_copy(rows_v.at[b, pl.ds(j * W, W)],
                         acc.at[ebuf.at[b, 1, j]], ssem[b], add=True)

    @pl.loop(q0, q1, step=NB)
    def _chunk_trip(i):
      for bb in range(NB):
        @pl.when(i + bb < q1)
        def _one():
          process(i + bb, bb)

    last = (q1 - 1 - q0) % NB
    for bb in range(NB):
      @pl.when(jnp.logical_or(last == bb, (last + NB - 1) % NB == bb))
      def _drain_tail():
        wait_scatter(bb)

    plsc.subcore_barrier()
    pltpu.sync_copy(acc.at[pl.ds(s * RPT, RPT)],
                    out_hbm.at[c, pl.ds(s * RPT, RPT)])

    @pl.when(s == NS - 1)
    def _write_tail():
      pltpu.sync_copy(acc.at[pl.ds(TAIL_START, TAIL)],
                      out_hbm.at[c, pl.ds(TAIL_START, TAIL)])

  return k


W = 80
_spmm_hid = _spmm_sc(NHID, 5, W)
_spmm_out = _spmm_sc(NCLASS, 16, W)


def _mm1_body(x_ref, w_ref, o_ref):
  o_ref[...] = jnp.dot(x_ref[...], w_ref[...],
                       preferred_element_type=jnp.float32)


def _mm1(x, W1):
  return pl.pallas_call(
      _mm1_body,
      out_shape=jax.ShapeDtypeStruct((N, NHID), jnp.float32),
  )(x, W1)


def _mid_body(p_ref, b1_ref, w2_ref, o_ref):
  h = p_ref[0] + p_ref[1] + b1_ref[...]
  h = jnp.maximum(h, 0.0)
  o_ref[...] = jnp.dot(h, w2_ref[...], preferred_element_type=jnp.float32)


def _mid(parts, b1, W2):
  return pl.pallas_call(
      _mid_body,
      out_shape=jax.ShapeDtypeStruct((N, NCLASS), jnp.float32),
  )(parts, b1, W2)


def _fin_body(q_ref, b2_ref, o_ref):
  o_ref[...] = q_ref[0] + q_ref[1] + b2_ref[...]


def _fin(parts, b2):
  return pl.pallas_call(
      _fin_body,
      out_shape=jax.ShapeDtypeStruct((N, NCLASS), jnp.float32),
  )(parts, b2)


def kernel(x, edge_index, edge_weight, W1, b1, W2, b2):
  ei = edge_index.astype(jnp.int32)
  wbits = lax.bitcast_convert_type(edge_weight.astype(jnp.float32), jnp.int32)
  epk = jnp.stack([ei[0].reshape(E // W, W), ei[1].reshape(E // W, W),
                   wbits.reshape(E // W, W)], axis=0)
  h = _mm1(x, W1)
  parts = _spmm_hid(epk, h)
  h2 = _mid(parts, b1.reshape(1, NHID), W2)
  parts2 = _spmm_out(epk, h2)
  return _fin(parts2, b2.reshape(1, NCLASS))


# speedup vs baseline: 1.5795x; 1.2102x over previous
import functools

import jax
import jax.numpy as jnp
from jax import lax
from jax.experimental import pallas as pl
from jax.experimental.pallas import tpu as pltpu
from jax.experimental.pallas import tpu_sc as plsc

N = 10000
E = 320000
NFEAT = 128
NHID = 64
NCLASS = 16

NC = 2
NS = 16
L = 16
NW = NC * NS
EPW = E // NW
RPT = 624
TAIL_START = NS * RPT
TAIL = N - TAIL_START


def _spmm_sc(feat: int, sb: int, w_dma: int):
  W = w_dma
  chunk = sb * W
  nch = E // chunk
  assert nch * chunk == E and W % 8 == 0 and W <= 128 and chunk % L == 0
  mesh = plsc.VectorSubcoreMesh(core_axis_name="c", subcore_axis_name="s")
  NB = 3
  NE = 4
  STEP = 12

  @functools.partial(
      pl.kernel,
      out_type=pltpu.HBM((NC, N, feat), jnp.float32),
      mesh=mesh,
      compiler_params=pltpu.CompilerParams(use_tc_tiling_on_sc=False),
      scratch_types=[
          pltpu.VMEM((NE, chunk), jnp.int32),
          pltpu.VMEM((NE, chunk), jnp.int32),
          pltpu.VMEM((NE, chunk), jnp.float32),
          pltpu.VMEM((NB, chunk, feat), jnp.float32),
          pltpu.VMEM_SHARED((N, feat), jnp.float32),
          [pltpu.SemaphoreType.DMA] * NE,
          [pltpu.SemaphoreType.DMA] * NB,
          [pltpu.SemaphoreType.DMA] * NB,
      ],
  )
  def k(ei_hbm, w_hbm, h_hbm, out_hbm, src_v, dst_v, w_v, rows_v,
        acc, esem, gsem, ssem):
    c = lax.axis_index("c")
    s = lax.axis_index("s")
    wid = s * NC + c

    q0 = wid * nch // NW
    q1 = (wid + 1) * nch // NW

    def fetch_edges(q, e):
      base = q * chunk
      pltpu.async_copy(ei_hbm.at[0, pl.ds(base, chunk)], src_v.at[e], esem[e])
      pltpu.async_copy(ei_hbm.at[1, pl.ds(base, chunk)], dst_v.at[e], esem[e])
      pltpu.async_copy(w_hbm.at[pl.ds(base, chunk)], w_v.at[e], esem[e])

    def wait_edges(q, e):
      base = q * chunk
      pltpu.make_async_copy(ei_hbm.at[0, pl.ds(base, chunk)], src_v.at[e],
                            esem[e]).wait()
      pltpu.make_async_copy(ei_hbm.at[1, pl.ds(base, chunk)], dst_v.at[e],
                            esem[e]).wait()
      pltpu.make_async_copy(w_hbm.at[pl.ds(base, chunk)], w_v.at[e],
                            esem[e]).wait()

    def fetch_gather(b, e):
      for j in range(sb):
        pltpu.async_copy(h_hbm.at[src_v.at[e, pl.ds(j * W, W)]],
                         rows_v.at[b, pl.ds(j * W, W)], gsem[b])

    def wait_gather(b, e):
      for j in range(sb):
        pltpu.make_async_copy(h_hbm.at[src_v.at[e, pl.ds(j * W, W)]],
                              rows_v.at[b, pl.ds(j * W, W)], gsem[b]).wait()

    def issue_scatter(b, e):
      for j in range(sb):
        pltpu.async_copy(rows_v.at[b, pl.ds(j * W, W)],
                         acc.at[dst_v.at[e, pl.ds(j * W, W)]], ssem[b],
                         add=True)

    def wait_scatter(b):
      for j in range(sb):
        pltpu.make_async_copy(rows_v.at[b, pl.ds(j * W, W)],
                              acc.at[dst_v.at[0, pl.ds(j * W, W)]],
                              ssem[b]).wait()

    fetch_edges(q0, 0)
    fetch_edges(q0 + 1, 1)
    wait_edges(q0, 0)
    fetch_gather(0, 0)

    zero = jnp.zeros((L,), jnp.float32)
    zrows = min(chunk, RPT)

    def zbody(i, _):
      for j in range(feat // L):
        rows_v[NB - 1, i, pl.ds(j * L, L)] = zero
      return 0

    lax.fori_loop(0, zrows, zbody, 0)
    done = 0
    while done < RPT:
      step = min(zrows, RPT - done)
      pltpu.sync_copy(rows_v.at[NB - 1, pl.ds(0, step)],
                      acc.at[pl.ds(s * RPT + done, step)])
      done += step

    @pl.when(s == NS - 1)
    def _zero_tail():
      pltpu.sync_copy(rows_v.at[NB - 1, pl.ds(0, TAIL)],
                      acc.at[pl.ds(TAIL_START, TAIL)])

    plsc.subcore_barrier()

    def process(q, b, e):
      wait_gather(b, e)
      nb = (b + 1) % NB
      ne1 = (e + 1) % NE
      ne2 = (e + 2) % NE

      @pl.when(q - 2 >= q0)
      def _drain_prev():
        wait_scatter(nb)

      @pl.when(q + 1 < q1)
      def _prefetch_gather():
        wait_edges(q + 1, ne1)
        fetch_gather(nb, ne1)

      @pl.when(q + 2 < q1)
      def _prefetch_edges():
        fetch_edges(q + 2, ne2)

      def gbody(g):
        w16 = w_v[e, pl.ds(g * L, L)]
        rowbase = g * L
        for ee in range(L):
          wb = w16[jnp.full((L,), ee, jnp.int32)]
          for f in range(feat // L):
            sl = pl.ds(f * L, L)
            rows_v[b, rowbase + ee, sl] = rows_v[b, rowbase + ee, sl] * wb

      plsc.parallel_loop(0, chunk // L, 1, unroll=4)(gbody)
      issue_scatter(b, e)

    @pl.loop(q0, q1, step=STEP)
    def _chunk_block(i):
      for kk in range(STEP):
        @pl.when(i + kk < q1)
        def _one():
          process(i + kk, kk % NB, kk % NE)

    last = (q1 - 1 - q0) % NB
    for bb in range(NB):
      @pl.when(jnp.logical_or(last == bb, (last + NB - 1) % NB == bb))
      def _drain_tail():
        wait_scatter(bb)

    plsc.subcore_barrier()
    pltpu.sync_copy(acc.at[pl.ds(s * RPT, RPT)],
                    out_hbm.at[c, pl.ds(s * RPT, RPT)])

    @pl.when(s == NS - 1)
    def _write_tail():
      pltpu.sync_copy(acc.at[pl.ds(TAIL_START, TAIL)],
                      out_hbm.at[c, pl.ds(TAIL_START, TAIL)])

  return k


_spmm_hid = _spmm_sc(NHID, 5, 80)
_spmm_out = _spmm_sc(NCLASS, 16, 80)


def _mm1_body(x_ref, w_ref, o_ref):
  o_ref[...] = jnp.dot(x_ref[...], w_ref[...],
                       preferred_element_type=jnp.float32)


def _mm1(x, W1):
  return pl.pallas_call(
      _mm1_body,
      out_shape=jax.ShapeDtypeStruct((N, NHID), jnp.float32),
  )(x, W1)


def _mid_body(p_ref, b1_ref, w2_ref, o_ref):
  h = p_ref[0] + p_ref[1] + b1_ref[...]
  h = jnp.maximum(h, 0.0)
  o_ref[...] = jnp.dot(h, w2_ref[...], preferred_element_type=jnp.float32)


def _mid(parts, b1, W2):
  return pl.pallas_call(
      _mid_body,
      out_shape=jax.ShapeDtypeStruct((N, NCLASS), jnp.float32),
  )(parts, b1, W2)


def _fin_body(q_ref, b2_ref, o_ref):
  o_ref[...] = q_ref[0] + q_ref[1] + b2_ref[...]


def _fin(parts, b2):
  return pl.pallas_call(
      _fin_body,
      out_shape=jax.ShapeDtypeStruct((N, NCLASS), jnp.float32),
  )(parts, b2)


def kernel(x, edge_index, edge_weight, W1, b1, W2, b2):
  ei = edge_index.astype(jnp.int32)
  w = edge_weight.astype(jnp.float32)
  h = _mm1(x, W1)
  parts = _spmm_hid(ei, w, h)
  h2 = _mid(parts, b1.reshape(1, NHID), W2)
  parts2 = _spmm_out(ei, w, h2)
  return _fin(parts2, b2.reshape(1, NCLASS))


# speedup vs baseline: 1.5926x; 1.0083x over previous
import functools

import jax
import jax.numpy as jnp
from jax import lax
from jax.experimental import pallas as pl
from jax.experimental.pallas import tpu as pltpu
from jax.experimental.pallas import tpu_sc as plsc

N = 10000
E = 320000
NFEAT = 128
NHID = 64
NCLASS = 16

NC = 2
NS = 16
L = 16
NW = NC * NS
EPW = E // NW
RPT = 624
TAIL_START = NS * RPT
TAIL = N - TAIL_START


def _spmm_sc(feat: int, sb: int, w_dma: int):
  W = w_dma
  chunk = sb * W
  nch = E // chunk
  assert nch * chunk == E and W % 8 == 0 and W <= 128 and chunk % L == 0
  mesh = plsc.VectorSubcoreMesh(core_axis_name="c", subcore_axis_name="s")
  NB = 3
  NE = 4
  STEP = 12

  @functools.partial(
      pl.kernel,
      out_type=pltpu.HBM((NC, N, feat), jnp.float32),
      mesh=mesh,
      compiler_params=pltpu.CompilerParams(use_tc_tiling_on_sc=False),
      scratch_types=[
          pltpu.VMEM((NE, chunk), jnp.int32),
          pltpu.VMEM((NE, chunk), jnp.int32),
          pltpu.VMEM((NE, chunk), jnp.float32),
          pltpu.VMEM((NB, chunk, feat), jnp.float32),
          pltpu.VMEM_SHARED((N, feat), jnp.float32),
          [pltpu.SemaphoreType.DMA] * NE,
          [pltpu.SemaphoreType.DMA] * NB,
          [pltpu.SemaphoreType.DMA] * NB,
      ],
  )
  def k(ei_hbm, w_hbm, h_hbm, out_hbm, src_v, dst_v, w_v, rows_v,
        acc, esem, gsem, ssem):
    c = lax.axis_index("c")
    s = lax.axis_index("s")
    wid = s * NC + c

    q0 = wid * nch // NW
    q1 = (wid + 1) * nch // NW

    def fetch_edges(q, e):
      base = q * chunk
      pltpu.async_copy(ei_hbm.at[0, pl.ds(base, chunk)], src_v.at[e], esem[e])
      pltpu.async_copy(ei_hbm.at[1, pl.ds(base, chunk)], dst_v.at[e], esem[e])
      pltpu.async_copy(w_hbm.at[pl.ds(base, chunk)], w_v.at[e], esem[e])

    def wait_edges(q, e):
      base = q * chunk
      pltpu.make_async_copy(ei_hbm.at[0, pl.ds(base, chunk)], src_v.at[e],
                            esem[e]).wait()
      pltpu.make_async_copy(ei_hbm.at[1, pl.ds(base, chunk)], dst_v.at[e],
                            esem[e]).wait()
      pltpu.make_async_copy(w_hbm.at[pl.ds(base, chunk)], w_v.at[e],
                            esem[e]).wait()

    def fetch_gather(b, e):
      for j in range(sb):
        pltpu.async_copy(h_hbm.at[src_v.at[e, pl.ds(j * W, W)]],
                         rows_v.at[b, pl.ds(j * W, W)], gsem[b])

    def wait_gather(b, e):
      for j in range(sb):
        pltpu.make_async_copy(h_hbm.at[src_v.at[e, pl.ds(j * W, W)]],
                              rows_v.at[b, pl.ds(j * W, W)], gsem[b]).wait()

    def issue_scatter(b, e):
      for j in range(sb):
        pltpu.async_copy(rows_v.at[b, pl.ds(j * W, W)],
                         acc.at[dst_v.at[e, pl.ds(j * W, W)]], ssem[b],
                         add=True)

    def wait_scatter(b):
      for j in range(sb):
        pltpu.make_async_copy(rows_v.at[b, pl.ds(j * W, W)],
                              acc.at[dst_v.at[0, pl.ds(j * W, W)]],
                              ssem[b]).wait()

    fetch_edges(q0, 0)
    fetch_edges(q0 + 1, 1)
    wait_edges(q0, 0)
    fetch_gather(0, 0)

    zero = jnp.zeros((L,), jnp.float32)
    zrows = min(chunk, RPT)

    def zbody(i, _):
      for j in range(feat // L):
        rows_v[NB - 1, i, pl.ds(j * L, L)] = zero
      return 0

    lax.fori_loop(0, zrows, zbody, 0)
    done = 0
    while done < RPT:
      step = min(zrows, RPT - done)
      pltpu.sync_copy(rows_v.at[NB - 1, pl.ds(0, step)],
                      acc.at[pl.ds(s * RPT + done, step)])
      done += step

    @pl.when(s == NS - 1)
    def _zero_tail():
      pltpu.sync_copy(rows_v.at[NB - 1, pl.ds(0, TAIL)],
                      acc.at[pl.ds(TAIL_START, TAIL)])

    plsc.subcore_barrier()

    def process(q, b, e):
      wait_gather(b, e)
      nb = (b + 1) % NB
      ne1 = (e + 1) % NE
      ne2 = (e + 2) % NE

      @pl.when(q - 2 >= q0)
      def _drain_prev():
        wait_scatter(nb)

      @pl.when(q + 1 < q1)
      def _prefetch_gather():
        wait_edges(q + 1, ne1)
        fetch_gather(nb, ne1)

      @pl.when(q + 2 < q1)
      def _prefetch_edges():
        fetch_edges(q + 2, ne2)

      def gbody(g):
        w16 = w_v[e, pl.ds(g * L, L)]
        rowbase = g * L
        for ee in range(L):
          wb = w16[jnp.full((L,), ee, jnp.int32)]
          for f in range(feat // L):
            sl = pl.ds(f * L, L)
            rows_v[b, rowbase + ee, sl] = rows_v[b, rowbase + ee, sl] * wb

      plsc.parallel_loop(0, chunk // L, 1, unroll=4)(gbody)
      issue_scatter(b, e)

    @pl.loop(q0, q1, step=STEP)
    def _chunk_block(i):
      for kk in range(STEP):
        @pl.when(i + kk < q1)
        def _one():
          process(i + kk, kk % NB, kk % NE)

    last = (q1 - 1 - q0) % NB
    for bb in range(NB):
      @pl.when(jnp.logical_or(last == bb, (last + NB - 1) % NB == bb))
      def _drain_tail():
        wait_scatter(bb)

    plsc.subcore_barrier()
    pltpu.sync_copy(acc.at[pl.ds(s * RPT, RPT)],
                    out_hbm.at[c, pl.ds(s * RPT, RPT)])

    @pl.when(s == NS - 1)
    def _write_tail():
      pltpu.sync_copy(acc.at[pl.ds(TAIL_START, TAIL)],
                      out_hbm.at[c, pl.ds(TAIL_START, TAIL)])

  return k


_spmm_hid = _spmm_sc(NHID, 5, 80)
_spmm_out = _spmm_sc(NCLASS, 10, 128)


def _mm1_body(x_ref, w_ref, o_ref):
  o_ref[...] = jnp.dot(x_ref[...], w_ref[...],
                       preferred_element_type=jnp.float32)


def _mm1(x, W1):
  return pl.pallas_call(
      _mm1_body,
      out_shape=jax.ShapeDtypeStruct((N, NHID), jnp.float32),
  )(x, W1)


def _mid_body(p_ref, b1_ref, w2_ref, o_ref):
  h = p_ref[0] + p_ref[1] + b1_ref[...]
  h = jnp.maximum(h, 0.0)
  o_ref[...] = jnp.dot(h, w2_ref[...], preferred_element_type=jnp.float32)


def _mid(parts, b1, W2):
  return pl.pallas_call(
      _mid_body,
      out_shape=jax.ShapeDtypeStruct((N, NCLASS), jnp.float32),
  )(parts, b1, W2)


def _fin_body(q_ref, b2_ref, o_ref):
  o_ref[...] = q_ref[0] + q_ref[1] + b2_ref[...]


def _fin(parts, b2):
  return pl.pallas_call(
      _fin_body,
      out_shape=jax.ShapeDtypeStruct((N, NCLASS), jnp.float32),
  )(parts, b2)


def kernel(x, edge_index, edge_weight, W1, b1, W2, b2):
  ei = edge_index.astype(jnp.int32)
  w = edge_weight.astype(jnp.float32)
  h = _mm1(x, W1)
  parts = _spmm_hid(ei, w, h)
  h2 = _mid(parts, b1.reshape(1, NHID), W2)
  parts2 = _spmm_out(ei, w, h2)
  return _fin(parts2, b2.reshape(1, NCLASS))
